# Initial kernel scaffold; baseline (speedup 1.0000x reference)
import jax
import jax.numpy as jnp
from jax.experimental import pallas as pl


def kernel(x, token_table, position_table):
    raise NotImplementedError("write your pallas kernel here")


# speedup vs baseline: 5.6560x; 5.6560x over previous
import functools

import jax
import jax.numpy as jnp
from jax import lax
from jax.experimental import pallas as pl
from jax.experimental.pallas import tpu as pltpu
from jax.experimental.pallas import tpu_sc as plsc

INPUT_DIM = 100000
OUTPUT_DIM = 32
BATCH = 16384
SEQ = 200

NC = 2
NS = 16
NW = NC * NS
L = 16

BB = 128
NBLK = BATCH // BB
BLK_PER_W = NBLK // NW
LHALF = SEQ // 2
DROWS = OUTPUT_DIM // 8
NROW = 4
NST = 2
ITEMS = LHALF * BLK_PER_W


@functools.partial(
    pl.kernel,
    out_type=jax.ShapeDtypeStruct((SEQ, DROWS, NBLK * 8 * 128), jnp.float32),
    mesh=plsc.VectorSubcoreMesh(core_axis_name="c", subcore_axis_name="s"),
    scratch_types=(
        [pltpu.VMEM((LHALF, BLK_PER_W, BB), jnp.int32)]
        + [pltpu.VMEM((BB, OUTPUT_DIM), jnp.float32) for _ in range(NROW)]
        + [pltpu.VMEM((DROWS * 1024,), jnp.float32) for _ in range(NST)]
        + [pltpu.VMEM((SEQ, OUTPUT_DIM), jnp.float32)]
        + [pltpu.SemaphoreType.DMA for _ in range(NROW + NST)]
    ),
    compiler_params=pltpu.CompilerParams(
        use_tc_tiling_on_sc=False, needs_layout_passes=False),
)
def _embed_kernel(x_hbm, table_hbm, pos_hbm, out_hbm, *refs):
    idx_v = refs[0]
    rows = refs[1:1 + NROW]
    stage = refs[1 + NROW:1 + NROW + NST]
    pos_v = refs[1 + NROW + NST]
    gsem = refs[2 + NROW + NST:2 + 2 * NROW + NST]
    osem = refs[2 + 2 * NROW + NST:]

    wid = lax.axis_index("s") * NC + lax.axis_index("c")
    wb0 = wid * BLK_PER_W

    pltpu.sync_copy(pos_hbm, pos_v)

    d0 = lax.iota(jnp.int32, L)
    base0 = lax.shift_left(lax.shift_right_logical(d0, 3), 10) + lax.shift_left(
        lax.rem(d0, jnp.int32(8)), 7)
    d1 = d0 + L
    base1 = lax.shift_left(lax.shift_right_logical(d1, 3), 10) + lax.shift_left(
        lax.rem(d1, jnp.int32(8)), 7)

    def fire_gather(li, b, r):
        pltpu.async_copy(table_hbm.at[idx_v.at[li, b]], rows[r], gsem[r])

    def wait_gather(r):
        pltpu.make_async_copy(
            table_hbm.at[idx_v.at[0, 0]], rows[r], gsem[r]).wait()

    def fire_out(l, bk, s):
        for k in range(DROWS):
            pltpu.async_copy(
                stage[s].at[pl.ds(k * 1024, 1024)],
                out_hbm.at[l, k, pl.ds(bk * 1024, 1024)],
                osem[s],
            )

    def wait_out(s):
        for k in range(DROWS):
            pltpu.make_async_copy(
                stage[s].at[pl.ds(k * 1024, 1024)],
                out_hbm.at[0, k, pl.ds(0, 1024)],
                osem[s],
            ).wait()

    def half_body(half):
        pltpu.sync_copy(
            x_hbm.at[pl.ds(half * LHALF, LHALF), pl.ds(wb0, BLK_PER_W)], idx_v)
        for m in range(NROW - 1):
            fire_gather(m // BLK_PER_W, m % BLK_PER_W, m)

        def outer(i, carry):
            for k in range(BLK_PER_W):
                r = k % NROW
                s = k % NST
                l = half * LHALF + i
                wait_gather(r)
                kn = (k + NROW - 1) % NROW
                lin = i + (k + NROW - 1) // NROW

                @pl.when(lin < LHALF)
                def _():
                    fire_gather(lin, kn, kn)

                m = i * BLK_PER_W + k

                @pl.when(m >= NST)
                def _():
                    wait_out(s)

                pl0 = pos_v[l, pl.ds(0, L)]
                pl1 = pos_v[l, pl.ds(L, L)]

                def tok_body(t, c):
                    v0 = rows[r][t, pl.ds(0, L)] + pl0
                    plsc.store_scatter(stage[s], [base0 + t], v0)
                    v1 = rows[r][t, pl.ds(L, L)] + pl1
                    plsc.store_scatter(stage[s], [base1 + t], v1)
                    return c

                lax.fori_loop(0, BB, tok_body, 0, unroll=8)
                fire_out(l, wb0 + k, s)
            return carry

        lax.fori_loop(0, LHALF, outer, 0, unroll=False)
        for m in range(ITEMS - NST, ITEMS):
            wait_out(m % NST)

    half_body(0)
    half_body(1)


def kernel(x, token_table, position_table):
    x_t = x.astype(jnp.int32).T.reshape(SEQ, NBLK, BB)
    pos = position_table[:SEQ]
    out = _embed_kernel(x_t, token_table, pos)
    out5 = out.reshape(SEQ, DROWS, NBLK, 8, 128)
    return out5.transpose(2, 4, 0, 1, 3).reshape(BATCH, SEQ, OUTPUT_DIM)
